# Initial kernel scaffold; baseline (speedup 1.0000x reference)
import jax
import jax.numpy as jnp
from jax.experimental import pallas as pl
from jax.experimental.pallas import tpu as pltpu

N_G = 262144
N_M = 128
K_TOP = int(N_G * 0.02)
ROWS = 8192
GRID = N_G // ROWS
OUT_R = N_G // 128


def _matvec_body(x_ref, w_ref, m_ref, o_ref):
    prod = w_ref[...] * m_ref[...] * x_ref[...]
    g = jnp.sum(prod, axis=1)
    o_ref[...] = g.reshape(ROWS // 128, 128)


def _select_body(g_ref, thr_ref, o_ref):
    g = g_ref[...]
    bits = jax.lax.bitcast_convert_type(g, jnp.int32)
    flip = jnp.where(bits < 0, jnp.uint32(0xFFFFFFFF), jnp.uint32(0x80000000))
    ukey = bits.astype(jnp.uint32) ^ flip

    def body(i, t):
        cand = t | (jnp.uint32(1) << (jnp.uint32(31) - i.astype(jnp.uint32)))
        cnt = jnp.sum((ukey >= cand).astype(jnp.int32))
        return jnp.where(cnt >= K_TOP, cand, t)

    t = jax.lax.fori_loop(0, 32, body, jnp.uint32(0))
    thr = thr_ref[0, 0]
    o_ref[...] = jnp.where(ukey >= t, jnp.maximum(g - thr, 0.0), 0.0)


def _build(interpret=False):
    matvec = pl.pallas_call(
        _matvec_body,
        grid=(GRID,),
        in_specs=[
            pl.BlockSpec((1, N_M), lambda i: (0, 0)),
            pl.BlockSpec((ROWS, N_M), lambda i: (i, 0)),
            pl.BlockSpec((ROWS, N_M), lambda i: (i, 0)),
        ],
        out_specs=pl.BlockSpec((ROWS // 128, 128), lambda i: (i, 0)),
        out_shape=jax.ShapeDtypeStruct((OUT_R, 128), jnp.float32),
        interpret=interpret,
    )
    select = pl.pallas_call(
        _select_body,
        out_shape=jax.ShapeDtypeStruct((OUT_R, 128), jnp.float32),
        interpret=interpret,
    )
    return matvec, select


_matvec, _select = _build()


def kernel(mossy_input, weights, connectivity_mask, threshold):
    x = mossy_input.reshape(1, N_M)
    g = _matvec(x, weights, connectivity_mask)
    thr = jnp.asarray(threshold, jnp.float32).reshape(1, 1)
    out = _select(g, thr)
    return out.reshape(N_G)


# speedup vs baseline: 3.1498x; 3.1498x over previous
import jax
import jax.numpy as jnp
from jax.experimental import pallas as pl
from jax.experimental.pallas import tpu as pltpu

N_G = 262144
N_M = 128
K_TOP = int(N_G * 0.02)
ROWS = 8192
GRID = N_G // ROWS
OUT_R = N_G // 128


def _matvec_body(x_ref, w_ref, m_ref, o_ref):
    mb = (w_ref[...] * m_ref[...]).astype(jnp.bfloat16).astype(jnp.float32)
    xb = x_ref[...].astype(jnp.bfloat16).astype(jnp.float32)
    g = jnp.sum(mb * xb, axis=1)
    o_ref[...] = g.reshape(ROWS // 128, 128)


def _select_body(g_ref, thr_ref, o_ref):
    g = g_ref[...]
    bits = jax.lax.bitcast_convert_type(g, jnp.int32)
    flip = jnp.where(bits < 0, jnp.uint32(0xFFFFFFFF), jnp.uint32(0x80000000))
    ukey = bits.astype(jnp.uint32) ^ flip

    def body(i, t):
        cand = t | (jnp.uint32(1) << (jnp.uint32(31) - i.astype(jnp.uint32)))
        cnt = jnp.sum((ukey >= cand).astype(jnp.int32))
        return jnp.where(cnt >= K_TOP, cand, t)

    t = jax.lax.fori_loop(0, 32, body, jnp.uint32(0))
    thr = thr_ref[0, 0]
    o_ref[...] = jnp.where(ukey >= t, jnp.maximum(g - thr, 0.0), 0.0)


def _build(interpret=False):
    matvec = pl.pallas_call(
        _matvec_body,
        grid=(GRID,),
        in_specs=[
            pl.BlockSpec((1, N_M), lambda i: (0, 0)),
            pl.BlockSpec((ROWS, N_M), lambda i: (i, 0)),
            pl.BlockSpec((ROWS, N_M), lambda i: (i, 0)),
        ],
        out_specs=pl.BlockSpec((ROWS // 128, 128), lambda i: (i, 0)),
        out_shape=jax.ShapeDtypeStruct((OUT_R, 128), jnp.float32),
        interpret=interpret,
    )
    select = pl.pallas_call(
        _select_body,
        out_shape=jax.ShapeDtypeStruct((OUT_R, 128), jnp.float32),
        interpret=interpret,
    )
    return matvec, select


_matvec, _select = _build()


def kernel(mossy_input, weights, connectivity_mask, threshold):
    x = mossy_input.reshape(1, N_M)
    g = _matvec(x, weights, connectivity_mask)
    thr = jnp.asarray(threshold, jnp.float32).reshape(1, 1)
    out = _select(g, thr)
    return out.reshape(N_G)
